# Initial kernel scaffold; baseline (speedup 1.0000x reference)
import jax
import jax.numpy as jnp
from jax.experimental import pallas as pl


def kernel(sequence_tensor, span_indices):
    raise NotImplementedError("write your pallas kernel here")


# speedup vs baseline: 1.0874x; 1.0874x over previous
import functools

import jax
import jax.numpy as jnp
from jax import lax
from jax.experimental import pallas as pl
from jax.experimental.pallas import tpu as pltpu
from jax.experimental.pallas import tpu_sc as plsc

B = 4
S = 4096
N = 2048
D = 768

ROWS = B * N * 2
NW = 32
RPW = ROWS // NW
C = 64
NCHUNK = RPW // C
L = 16

_mesh = plsc.VectorSubcoreMesh(core_axis_name="c", subcore_axis_name="s")


@functools.partial(
    pl.kernel,
    mesh=_mesh,
    out_type=jax.ShapeDtypeStruct((ROWS, D), jnp.float32),
    scratch_types=[
        pltpu.VMEM((RPW,), jnp.int32),
        pltpu.VMEM((C, D), jnp.float32),
        pltpu.VMEM((C, D), jnp.float32),
        pltpu.SemaphoreType.DMA,
        pltpu.SemaphoreType.DMA,
    ],
)
def _sc_gather(table_hbm, idx_hbm, out_hbm, idx_v, buf0, buf1, sem0, sem1):
    wid = lax.axis_index("s") * 2 + lax.axis_index("c")
    base = wid * RPW
    pltpu.sync_copy(idx_hbm.at[pl.ds(base, RPW)], idx_v)
    boff = (base // (2 * N)) * S
    for i in range(RPW // L):
        idx_v[pl.ds(i * L, L)] = idx_v[pl.ds(i * L, L)] + boff

    bufs = (buf0, buf1)
    sems = (sem0, sem1)

    def start_gather(ci):
        return pltpu.async_copy(
            table_hbm.at[idx_v.at[pl.ds(ci * C, C)]], bufs[ci % 2], sems[ci % 2]
        )

    pending = start_gather(0)
    for ci in range(NCHUNK):
        cur = pending
        if ci + 1 < NCHUNK:
            pending = start_gather(ci + 1)
        cur.wait()
        pltpu.sync_copy(bufs[ci % 2], out_hbm.at[pl.ds(base + ci * C, C)])


def kernel(sequence_tensor, span_indices):
    table = sequence_tensor.reshape(B * S, D)
    idx = span_indices.astype(jnp.int32).reshape(ROWS)
    out = _sc_gather(table, idx)
    return out.reshape(B, N, 2 * D)


# speedup vs baseline: 2.1390x; 1.9670x over previous
import functools

import jax
import jax.numpy as jnp
from jax import lax
from jax.experimental import pallas as pl
from jax.experimental.pallas import tpu as pltpu
from jax.experimental.pallas import tpu_sc as plsc

B = 4
S = 4096
N = 2048
D = 768

NW = 32
SPW = B * N // NW
CS = 32
NCHUNK = SPW // CS
L = 16

_mesh = plsc.VectorSubcoreMesh(core_axis_name="c", subcore_axis_name="s")


@functools.partial(
    pl.kernel,
    mesh=_mesh,
    out_type=jax.ShapeDtypeStruct((B, N, 2 * D), jnp.float32),
    scratch_types=[
        pltpu.VMEM((SPW,), jnp.int32),
        pltpu.VMEM((SPW,), jnp.int32),
        pltpu.VMEM((CS, D), jnp.float32),
        pltpu.VMEM((CS, D), jnp.float32),
        pltpu.VMEM((CS, D), jnp.float32),
        pltpu.VMEM((CS, D), jnp.float32),
        pltpu.SemaphoreType.DMA,
        pltpu.SemaphoreType.DMA,
        pltpu.SemaphoreType.DMA,
        pltpu.SemaphoreType.DMA,
    ],
)
def _sc_gather(
    table_hbm, sidx_hbm, eidx_hbm, out_hbm,
    idx_s, idx_e, bs0, bs1, be0, be1, sem_s0, sem_s1, sem_e0, sem_e1,
):
    wid = lax.axis_index("s") * 2 + lax.axis_index("c")
    sbase = wid * SPW
    b = sbase // N
    nb = sbase - b * N
    pltpu.sync_copy(sidx_hbm.at[pl.ds(sbase, SPW)], idx_s)
    pltpu.sync_copy(eidx_hbm.at[pl.ds(sbase, SPW)], idx_e)
    boff = b * S
    for i in range(SPW // L):
        idx_s[pl.ds(i * L, L)] = idx_s[pl.ds(i * L, L)] + boff
        idx_e[pl.ds(i * L, L)] = idx_e[pl.ds(i * L, L)] + boff

    bufs_s = (bs0, bs1)
    bufs_e = (be0, be1)
    sems_s = (sem_s0, sem_s1)
    sems_e = (sem_e0, sem_e1)

    def start_gathers(ci):
        k = ci % 2
        hs = pltpu.async_copy(
            table_hbm.at[idx_s.at[pl.ds(ci * CS, CS)]], bufs_s[k], sems_s[k]
        )
        he = pltpu.async_copy(
            table_hbm.at[idx_e.at[pl.ds(ci * CS, CS)]], bufs_e[k], sems_e[k]
        )
        return hs, he

    pending = start_gathers(0)
    for ci in range(NCHUNK):
        cur_s, cur_e = pending
        if ci + 1 < NCHUNK:
            pending = start_gathers(ci + 1)
        cur_s.wait()
        cur_e.wait()
        k = ci % 2
        row0 = nb + ci * CS
        pltpu.sync_copy(bufs_s[k], out_hbm.at[b, pl.ds(row0, CS), pl.ds(0, D)])
        pltpu.sync_copy(bufs_e[k], out_hbm.at[b, pl.ds(row0, CS), pl.ds(D, D)])


def kernel(sequence_tensor, span_indices):
    table = sequence_tensor.reshape(B * S, D)
    si = span_indices.astype(jnp.int32)
    starts = si[..., 0].reshape(B * N)
    ends = si[..., 1].reshape(B * N)
    return _sc_gather(table, starts, ends)


# speedup vs baseline: 2.1503x; 1.0053x over previous
import functools

import jax
import jax.numpy as jnp
from jax import lax
from jax.experimental import pallas as pl
from jax.experimental.pallas import tpu as pltpu
from jax.experimental.pallas import tpu_sc as plsc

B = 4
S = 4096
N = 2048
D = 768

NW = 32
SPW = B * N // NW
CS = 32
NCHUNK = SPW // CS
L = 16

_mesh = plsc.VectorSubcoreMesh(core_axis_name="c", subcore_axis_name="s")


@functools.partial(
    pl.kernel,
    mesh=_mesh,
    out_type=jax.ShapeDtypeStruct((B, N, 2 * D), jnp.float32),
    scratch_types=[
        pltpu.VMEM((SPW,), jnp.int32),
        pltpu.VMEM((SPW,), jnp.int32),
        pltpu.VMEM((CS, D), jnp.float32),
        pltpu.VMEM((CS, D), jnp.float32),
        pltpu.VMEM((CS, D), jnp.float32),
        pltpu.VMEM((CS, D), jnp.float32),
        pltpu.SemaphoreType.DMA,
        pltpu.SemaphoreType.DMA,
        pltpu.SemaphoreType.DMA,
        pltpu.SemaphoreType.DMA,
        pltpu.SemaphoreType.DMA,
        pltpu.SemaphoreType.DMA,
        pltpu.SemaphoreType.DMA,
        pltpu.SemaphoreType.DMA,
    ],
)
def _sc_gather(
    table_hbm, sidx_hbm, eidx_hbm, out_hbm,
    idx_s, idx_e, bs0, bs1, be0, be1,
    sem_s0, sem_s1, sem_e0, sem_e1, wsem_s0, wsem_s1, wsem_e0, wsem_e1,
):
    wid = lax.axis_index("s") * 2 + lax.axis_index("c")
    sbase = wid * SPW
    b = sbase // N
    nb = sbase - b * N
    pltpu.sync_copy(sidx_hbm.at[pl.ds(sbase, SPW)], idx_s)
    pltpu.sync_copy(eidx_hbm.at[pl.ds(sbase, SPW)], idx_e)
    boff = b * S
    for i in range(SPW // L):
        idx_s[pl.ds(i * L, L)] = idx_s[pl.ds(i * L, L)] + boff
        idx_e[pl.ds(i * L, L)] = idx_e[pl.ds(i * L, L)] + boff

    bufs_s = (bs0, bs1)
    bufs_e = (be0, be1)
    sems_s = (sem_s0, sem_s1)
    sems_e = (sem_e0, sem_e1)
    wsems_s = (wsem_s0, wsem_s1)
    wsems_e = (wsem_e0, wsem_e1)

    def start_gathers(ci):
        k = ci % 2
        hs = pltpu.async_copy(
            table_hbm.at[idx_s.at[pl.ds(ci * CS, CS)]], bufs_s[k], sems_s[k]
        )
        he = pltpu.async_copy(
            table_hbm.at[idx_e.at[pl.ds(ci * CS, CS)]], bufs_e[k], sems_e[k]
        )
        return hs, he

    writes = {}
    pending = start_gathers(0)
    for ci in range(NCHUNK):
        cur_s, cur_e = pending
        if ci + 1 < NCHUNK:
            if ci - 1 in writes:
                for h in writes.pop(ci - 1):
                    h.wait()
            pending = start_gathers(ci + 1)
        cur_s.wait()
        cur_e.wait()
        k = ci % 2
        row0 = nb + ci * CS
        writes[ci] = (
            pltpu.async_copy(
                bufs_s[k], out_hbm.at[b, pl.ds(row0, CS), pl.ds(0, D)], wsems_s[k]
            ),
            pltpu.async_copy(
                bufs_e[k], out_hbm.at[b, pl.ds(row0, CS), pl.ds(D, D)], wsems_e[k]
            ),
        )
    for ci in sorted(writes):
        for h in writes[ci]:
            h.wait()


def kernel(sequence_tensor, span_indices):
    table = sequence_tensor.reshape(B * S, D)
    si = span_indices.astype(jnp.int32)
    starts = si[..., 0].reshape(B * N)
    ends = si[..., 1].reshape(B * N)
    return _sc_gather(table, starts, ends)
